# Initial kernel scaffold; baseline (speedup 1.0000x reference)
import jax
import jax.numpy as jnp
from jax.experimental import pallas as pl


def kernel(node_ft, batch):
    raise NotImplementedError("write your pallas kernel here")


# speedup vs baseline: 1.8913x; 1.8913x over previous
import functools

import jax
import jax.numpy as jnp
from jax import lax
from jax.experimental import pallas as pl
from jax.experimental.pallas import tpu as pltpu
from jax.experimental.pallas import tpu_sc as plsc

S = 10000
N = 320000
D = 128
NC = 1
NW = 16 * NC
ROWS_PER_W = N // NW
B = 80
NBLK = ROWS_PER_W // B
SP = 10240
S_PER_TILE = SP // 16
ZCH = S_PER_TILE // B


def _sc_body(node, batch, zrow, onerow, acc_o, idx_v, buf, zbuf, obuf,
             acc_sh):
    c = lax.axis_index("c")
    s = lax.axis_index("s")
    row0 = (s * NC + c) * ROWS_PER_W
    pltpu.sync_copy(zrow, zbuf)
    pltpu.sync_copy(onerow, obuf)
    for j in range(ZCH):
        pltpu.sync_copy(zbuf, acc_sh.at[pl.ds(s * S_PER_TILE + j * B, B), :])
    plsc.subcore_barrier()

    def blk(k, carry):
        pltpu.sync_copy(batch.at[pl.ds(row0 + k * B, B)], idx_v)
        pltpu.sync_copy(node.at[pl.ds(row0 + k * B, B), :], buf)
        pltpu.sync_copy(buf, acc_sh.at[idx_v], add=True)
        return carry

    lax.fori_loop(0, NBLK, blk, 0)
    plsc.subcore_barrier()
    for j in range(ZCH):
        base = s * S_PER_TILE + j * B
        pltpu.sync_copy(acc_sh.at[pl.ds(base, B), :], buf)
        pltpu.sync_copy(buf, acc_o.at[c, 0, pl.ds(base, B), :])
        pltpu.sync_copy(zbuf, acc_sh.at[pl.ds(base, B), :])
    plsc.subcore_barrier()

    def cblk(k, carry):
        pltpu.sync_copy(batch.at[pl.ds(row0 + k * B, B)], idx_v)
        pltpu.sync_copy(obuf, acc_sh.at[idx_v], add=True)
        return carry

    lax.fori_loop(0, NBLK, cblk, 0)
    plsc.subcore_barrier()
    for j in range(ZCH):
        base = s * S_PER_TILE + j * B
        pltpu.sync_copy(acc_sh.at[pl.ds(base, B), :], buf)
        pltpu.sync_copy(buf, acc_o.at[c, 1, pl.ds(base, B), :])


_sc_accumulate = functools.partial(
    pl.kernel,
    out_type=jax.ShapeDtypeStruct((NC, 2, SP, D), jnp.float32),
    mesh=plsc.VectorSubcoreMesh(core_axis_name="c", subcore_axis_name="s",
                                num_cores=NC),
    scratch_types=[
        pltpu.VMEM((B,), jnp.int32),
        pltpu.VMEM((B, D), jnp.float32),
        pltpu.VMEM((B, D), jnp.float32),
        pltpu.VMEM((B, D), jnp.float32),
        pltpu.VMEM_SHARED((SP, D), jnp.float32),
    ],
)(_sc_body)


def _combine_body(pa_ref, o_ref):
    sm = pa_ref[0, 0]
    cn = pa_ref[0, 1, :, 0:1]
    for k in range(1, NC):
        sm = sm + pa_ref[k, 0]
        cn = cn + pa_ref[k, 1, :, 0:1]
    o_ref[...] = sm / jnp.maximum(cn, 1.0)


def _combine(pacc):
    blk = 1000
    return pl.pallas_call(
        _combine_body,
        grid=(S // blk,),
        in_specs=[pl.BlockSpec((NC, 2, blk, D), lambda i: (0, 0, i, 0))],
        out_specs=pl.BlockSpec((blk, D), lambda i: (i, 0)),
        out_shape=jax.ShapeDtypeStruct((S, D), jnp.float32),
    )(pacc)


def kernel(node_ft, batch):
    zrow = jnp.zeros((B, D), jnp.float32)
    onerow = jnp.ones((B, D), jnp.float32)
    pacc = _sc_accumulate(node_ft, batch, zrow, onerow)
    return _combine(pacc)


# speedup vs baseline: 3.0543x; 1.6149x over previous
import functools

import jax
import jax.numpy as jnp
from jax import lax
from jax.experimental import pallas as pl
from jax.experimental.pallas import tpu as pltpu
from jax.experimental.pallas import tpu_sc as plsc

S = 10000
N = 320000
D = 128
NC = 1
NW = 16 * NC
ROWS_PER_W = N // NW
B = 80
NBLK = ROWS_PER_W // B
NB2 = NBLK // 2
SP = 10240
S_PER_TILE = SP // 16
ZCH = S_PER_TILE // B


def _sc_body(node, batch, zrow, onerow, acc_o,
             idx0, idx1, buf0, buf1, zbuf, obuf, acc_sh,
             si0, si1, sr0, sr1, sc0, sc1):
    c = lax.axis_index("c")
    s = lax.axis_index("s")
    row0 = (s * NC + c) * ROWS_PER_W

    def idx_at(k):
        return batch.at[pl.ds(row0 + k * B, B)]

    def rows_at(k):
        return node.at[pl.ds(row0 + k * B, B), :]

    pltpu.sync_copy(zrow, zbuf)
    pltpu.sync_copy(onerow, obuf)
    for j in range(ZCH):
        pltpu.sync_copy(zbuf, acc_sh.at[pl.ds(s * S_PER_TILE + j * B, B), :])
    plsc.subcore_barrier()

    pltpu.async_copy(idx_at(0), idx0, si0)
    pltpu.async_copy(rows_at(0), buf0, sr0)

    def blk(k2, carry):
        k = 2 * k2

        @pl.when(k2 > 0)
        def _():
            pltpu.make_async_copy(buf1, acc_sh.at[idx1], sc1).wait()

        pltpu.async_copy(idx_at(k + 1), idx1, si1)
        pltpu.async_copy(rows_at(k + 1), buf1, sr1)
        pltpu.make_async_copy(idx_at(k), idx0, si0).wait()
        pltpu.make_async_copy(rows_at(k), buf0, sr0).wait()
        pltpu.async_copy(buf0, acc_sh.at[idx0], sc0, add=True)
        pltpu.make_async_copy(idx_at(k + 1), idx1, si1).wait()
        pltpu.make_async_copy(rows_at(k + 1), buf1, sr1).wait()
        pltpu.async_copy(buf1, acc_sh.at[idx1], sc1, add=True)

        @pl.when(k2 < NB2 - 1)
        def _():
            pltpu.make_async_copy(buf0, acc_sh.at[idx0], sc0).wait()
            pltpu.async_copy(idx_at(k + 2), idx0, si0)
            pltpu.async_copy(rows_at(k + 2), buf0, sr0)

        return carry

    lax.fori_loop(0, NB2, blk, 0)
    pltpu.make_async_copy(buf0, acc_sh.at[idx0], sc0).wait()
    pltpu.make_async_copy(buf1, acc_sh.at[idx1], sc1).wait()
    plsc.subcore_barrier()

    for j in range(ZCH):
        base = s * S_PER_TILE + j * B
        pltpu.sync_copy(acc_sh.at[pl.ds(base, B), :], buf0)
        pltpu.sync_copy(buf0, acc_o.at[c, 0, pl.ds(base, B), :])
        pltpu.sync_copy(zbuf, acc_sh.at[pl.ds(base, B), :])
    plsc.subcore_barrier()

    pltpu.async_copy(idx_at(0), idx0, si0)

    def cblk(k2, carry):
        k = 2 * k2

        @pl.when(k2 > 0)
        def _():
            pltpu.make_async_copy(obuf, acc_sh.at[idx1], sc1).wait()

        pltpu.async_copy(idx_at(k + 1), idx1, si1)
        pltpu.make_async_copy(idx_at(k), idx0, si0).wait()
        pltpu.async_copy(obuf, acc_sh.at[idx0], sc0, add=True)
        pltpu.make_async_copy(idx_at(k + 1), idx1, si1).wait()
        pltpu.async_copy(obuf, acc_sh.at[idx1], sc1, add=True)

        @pl.when(k2 < NB2 - 1)
        def _():
            pltpu.make_async_copy(obuf, acc_sh.at[idx0], sc0).wait()
            pltpu.async_copy(idx_at(k + 2), idx0, si0)

        return carry

    lax.fori_loop(0, NB2, cblk, 0)
    pltpu.make_async_copy(obuf, acc_sh.at[idx0], sc0).wait()
    pltpu.make_async_copy(obuf, acc_sh.at[idx1], sc1).wait()
    plsc.subcore_barrier()
    for j in range(ZCH):
        base = s * S_PER_TILE + j * B
        pltpu.sync_copy(acc_sh.at[pl.ds(base, B), :], buf0)
        pltpu.sync_copy(buf0, acc_o.at[c, 1, pl.ds(base, B), :])


_sc_accumulate = functools.partial(
    pl.kernel,
    out_type=jax.ShapeDtypeStruct((NC, 2, SP, D), jnp.float32),
    mesh=plsc.VectorSubcoreMesh(core_axis_name="c", subcore_axis_name="s",
                                num_cores=NC),
    scratch_types=[
        pltpu.VMEM((B,), jnp.int32),
        pltpu.VMEM((B,), jnp.int32),
        pltpu.VMEM((B, D), jnp.float32),
        pltpu.VMEM((B, D), jnp.float32),
        pltpu.VMEM((B, D), jnp.float32),
        pltpu.VMEM((B, D), jnp.float32),
        pltpu.VMEM_SHARED((SP, D), jnp.float32),
        pltpu.SemaphoreType.DMA,
        pltpu.SemaphoreType.DMA,
        pltpu.SemaphoreType.DMA,
        pltpu.SemaphoreType.DMA,
        pltpu.SemaphoreType.DMA,
        pltpu.SemaphoreType.DMA,
    ],
)(_sc_body)


def _combine_body(pa_ref, o_ref):
    sm = pa_ref[0, 0]
    cn = pa_ref[0, 1, :, 0:1]
    for k in range(1, NC):
        sm = sm + pa_ref[k, 0]
        cn = cn + pa_ref[k, 1, :, 0:1]
    o_ref[...] = sm / jnp.maximum(cn, 1.0)


def _combine(pacc):
    blk = 1000
    return pl.pallas_call(
        _combine_body,
        grid=(S // blk,),
        in_specs=[pl.BlockSpec((NC, 2, blk, D), lambda i: (0, 0, i, 0))],
        out_specs=pl.BlockSpec((blk, D), lambda i: (i, 0)),
        out_shape=jax.ShapeDtypeStruct((S, D), jnp.float32),
    )(pacc)


def kernel(node_ft, batch):
    zrow = jnp.zeros((B, D), jnp.float32)
    onerow = jnp.ones((B, D), jnp.float32)
    pacc = _sc_accumulate(node_ft, batch, zrow, onerow)
    return _combine(pacc)


# speedup vs baseline: 3.8354x; 1.2557x over previous
import functools

import jax
import jax.numpy as jnp
from jax import lax
from jax.experimental import pallas as pl
from jax.experimental.pallas import tpu as pltpu
from jax.experimental.pallas import tpu_sc as plsc

S = 10000
N = 320000
D = 128
NC = 1
NW = 16 * NC
ROWS_PER_W = N // NW
B = 40
NBLK = ROWS_PER_W // B
NSLOT = 5
HALF = NSLOT // 2
NOUT = NBLK // NSLOT
SP = 10240
S_PER_TILE = SP // 16
ZCH = S_PER_TILE // B


def _sc_body(node, batch, zrow, onerow, acc_o, *refs):
    idxs = refs[0:NSLOT]
    bufs = refs[NSLOT:2 * NSLOT]
    obuf = refs[2 * NSLOT]
    acc_sh = refs[2 * NSLOT + 1]
    gsem = refs[2 * NSLOT + 2:2 * NSLOT + 2 + NSLOT]
    ssem = refs[2 * NSLOT + 2 + NSLOT:2 * NSLOT + 2 + 2 * NSLOT]

    c = lax.axis_index("c")
    s = lax.axis_index("s")
    row0 = (s * NC + c) * ROWS_PER_W

    def idx_at(k):
        return batch.at[pl.ds(row0 + k * B, B)]

    def rows_at(k):
        return node.at[pl.ds(row0 + k * B, B), :]

    pltpu.sync_copy(zrow, bufs[0])
    pltpu.sync_copy(onerow, obuf)
    for j in range(ZCH):
        pltpu.sync_copy(bufs[0], acc_sh.at[pl.ds(s * S_PER_TILE + j * B, B), :])
    plsc.subcore_barrier()

    for t in range(HALF):
        pltpu.async_copy(idx_at(t), idxs[t], gsem[t])
        pltpu.async_copy(rows_at(t), bufs[t], gsem[t])

    def blk(g, carry):
        k0 = g * NSLOT
        for t in range(HALF, NSLOT):
            @pl.when(g > 0)
            def _(t=t):
                pltpu.make_async_copy(bufs[t], acc_sh.at[idxs[t]],
                                      ssem[t]).wait()
            pltpu.async_copy(idx_at(k0 + t), idxs[t], gsem[t])
            pltpu.async_copy(rows_at(k0 + t), bufs[t], gsem[t])
        for t in range(HALF):
            pltpu.make_async_copy(idx_at(k0 + t), idxs[t], gsem[t]).wait()
            pltpu.make_async_copy(rows_at(k0 + t), bufs[t], gsem[t]).wait()
            pltpu.async_copy(bufs[t], acc_sh.at[idxs[t]], ssem[t], add=True)
        for t in range(HALF, NSLOT):
            pltpu.make_async_copy(idx_at(k0 + t), idxs[t], gsem[t]).wait()
            pltpu.make_async_copy(rows_at(k0 + t), bufs[t], gsem[t]).wait()
            pltpu.async_copy(bufs[t], acc_sh.at[idxs[t]], ssem[t], add=True)
        for t in range(HALF):
            @pl.when(g < NOUT - 1)
            def _(t=t):
                pltpu.make_async_copy(bufs[t], acc_sh.at[idxs[t]],
                                      ssem[t]).wait()
                pltpu.async_copy(idx_at(k0 + NSLOT + t), idxs[t], gsem[t])
                pltpu.async_copy(rows_at(k0 + NSLOT + t), bufs[t], gsem[t])
        return carry

    lax.fori_loop(0, NOUT, blk, 0)
    for t in range(HALF):
        pltpu.make_async_copy(bufs[t], acc_sh.at[idxs[t]], ssem[t]).wait()
    for t in range(HALF, NSLOT):
        pltpu.make_async_copy(bufs[t], acc_sh.at[idxs[t]], ssem[t]).wait()
    plsc.subcore_barrier()

    pltpu.sync_copy(zrow, bufs[1])
    for j in range(ZCH):
        base = s * S_PER_TILE + j * B
        pltpu.sync_copy(acc_sh.at[pl.ds(base, B), :], bufs[0])
        pltpu.sync_copy(bufs[0], acc_o.at[c, 0, pl.ds(base, B), :])
        pltpu.sync_copy(bufs[1], acc_sh.at[pl.ds(base, B), :])
    plsc.subcore_barrier()

    for t in range(HALF):
        pltpu.async_copy(idx_at(t), idxs[t], gsem[t])

    def cblk(g, carry):
        k0 = g * NSLOT
        for t in range(HALF, NSLOT):
            @pl.when(g > 0)
            def _(t=t):
                pltpu.make_async_copy(obuf, acc_sh.at[idxs[t]],
                                      ssem[t]).wait()
            pltpu.async_copy(idx_at(k0 + t), idxs[t], gsem[t])
        for t in range(HALF):
            pltpu.make_async_copy(idx_at(k0 + t), idxs[t], gsem[t]).wait()
            pltpu.async_copy(obuf, acc_sh.at[idxs[t]], ssem[t], add=True)
        for t in range(HALF, NSLOT):
            pltpu.make_async_copy(idx_at(k0 + t), idxs[t], gsem[t]).wait()
            pltpu.async_copy(obuf, acc_sh.at[idxs[t]], ssem[t], add=True)
        for t in range(HALF):
            @pl.when(g < NOUT - 1)
            def _(t=t):
                pltpu.make_async_copy(obuf, acc_sh.at[idxs[t]],
                                      ssem[t]).wait()
                pltpu.async_copy(idx_at(k0 + NSLOT + t), idxs[t], gsem[t])
        return carry

    lax.fori_loop(0, NOUT, cblk, 0)
    for t in range(HALF):
        pltpu.make_async_copy(obuf, acc_sh.at[idxs[t]], ssem[t]).wait()
    for t in range(HALF, NSLOT):
        pltpu.make_async_copy(obuf, acc_sh.at[idxs[t]], ssem[t]).wait()
    plsc.subcore_barrier()
    for j in range(ZCH):
        base = s * S_PER_TILE + j * B
        pltpu.sync_copy(acc_sh.at[pl.ds(base, B), :], bufs[0])
        pltpu.sync_copy(bufs[0], acc_o.at[c, 1, pl.ds(base, B), :])


_sc_accumulate = functools.partial(
    pl.kernel,
    out_type=jax.ShapeDtypeStruct((NC, 2, SP, D), jnp.float32),
    mesh=plsc.VectorSubcoreMesh(core_axis_name="c", subcore_axis_name="s",
                                num_cores=NC),
    scratch_types=(
        [pltpu.VMEM((B,), jnp.int32) for _ in range(NSLOT)]
        + [pltpu.VMEM((B, D), jnp.float32) for _ in range(NSLOT)]
        + [pltpu.VMEM((B, D), jnp.float32)]
        + [pltpu.VMEM_SHARED((SP, D), jnp.float32)]
        + [pltpu.SemaphoreType.DMA for _ in range(2 * NSLOT)]
    ),
)(_sc_body)


def _combine_body(pa_ref, o_ref):
    sm = pa_ref[0, 0]
    cn = pa_ref[0, 1, :, 0:1]
    for k in range(1, NC):
        sm = sm + pa_ref[k, 0]
        cn = cn + pa_ref[k, 1, :, 0:1]
    o_ref[...] = sm / jnp.maximum(cn, 1.0)


def _combine(pacc):
    blk = 1000
    return pl.pallas_call(
        _combine_body,
        grid=(S // blk,),
        in_specs=[pl.BlockSpec((NC, 2, blk, D), lambda i: (0, 0, i, 0))],
        out_specs=pl.BlockSpec((blk, D), lambda i: (i, 0)),
        out_shape=jax.ShapeDtypeStruct((S, D), jnp.float32),
    )(pacc)


def kernel(node_ft, batch):
    zrow = jnp.zeros((B, D), jnp.float32)
    onerow = jnp.ones((B, D), jnp.float32)
    pacc = _sc_accumulate(node_ft, batch, zrow, onerow)
    return _combine(pacc)
